# baseline (speedup 1.0000x reference)
import functools

import jax
import jax.numpy as jnp
from jax import lax
from jax.experimental import pallas as pl
from jax.experimental.pallas import tpu as pltpu
from jax.experimental.pallas import tpu_sc as plsc

_EPS = 1e-12
_NC, _NS = 2, 16
_NW = _NC * _NS
_LANES = 16


_CH = 32


def _sc_gather_sum(word_emb, x_emb, y_emb, h_emb, w_emb, idx5c, tok):
    hdim = word_emb.shape[1]
    b_per_w = tok // _NW
    ch = _CH
    n_chunks = b_per_w // ch
    assert tok % _NW == 0 and b_per_w % (2 * ch) == 0 and hdim % _LANES == 0

    mesh = plsc.VectorSubcoreMesh(
        core_axis_name="c", subcore_axis_name="s",
        num_cores=_NC, num_subcores=_NS)

    @functools.partial(
        pl.kernel,
        out_type=jax.ShapeDtypeStruct((tok, hdim), jnp.float32),
        mesh=mesh,
        scratch_types=[
            pltpu.VMEM((8 * ch,), jnp.int32),
            pltpu.VMEM((8 * ch,), jnp.int32),
            pltpu.VMEM((ch, hdim), jnp.float32),
            pltpu.VMEM((ch, hdim), jnp.float32),
            pltpu.VMEM((ch, hdim), jnp.float32),
            pltpu.VMEM((ch, hdim), jnp.float32),
            pltpu.SemaphoreType.DMA,
            pltpu.SemaphoreType.DMA,
            pltpu.SemaphoreType.DMA,
            pltpu.SemaphoreType.DMA,
            pltpu.SemaphoreType.DMA,
            pltpu.SemaphoreType.DMA,
        ],
    )
    def k(word_hbm, x_hbm, y_hbm, h_hbm, w_hbm, idx5_hbm, out_hbm,
          idx0, idx1, acc0, acc1, buf_a, buf_b,
          sem_i, sem_w, sem_a, sem_b, sem_o0, sem_o1):
        wid = lax.axis_index("s") * _NC + lax.axis_index("c")
        w0 = wid * b_per_w
        t0 = wid * n_chunks

        def fetch_idx(c, idxb):
            return pltpu.async_copy(idx5_hbm.at[t0 + c], idxb, sem_i)

        fetch_idx(0, idx0)

        def chunk_body(c, idxb, other_idxb, acc, sem_o, last_parity):
            base = w0 + c * ch

            def grp(j):
                return idxb.at[pl.ds(j * ch, ch)]

            pltpu.make_async_copy(idx5_hbm.at[t0 + c], idxb, sem_i).wait()
            if not last_parity:
                fetch_idx(c + 1, other_idxb)
            else:
                @pl.when(c + 1 < n_chunks)
                def _():
                    fetch_idx(c + 1, other_idxb)
            for i in range(0, ch, _LANES):
                idxb[pl.ds(5 * ch + i, _LANES)] = (
                    idxb[pl.ds(4 * ch + i, _LANES)]
                    - idxb[pl.ds(2 * ch + i, _LANES)])
                idxb[pl.ds(6 * ch + i, _LANES)] = (
                    idxb[pl.ds(3 * ch + i, _LANES)]
                    - idxb[pl.ds(1 * ch + i, _LANES)])

            tabs = (x_hbm, y_hbm, x_hbm, h_hbm, w_hbm, y_hbm)
            irow = (1, 2, 3, 5, 6, 4)
            bufs = (buf_a, buf_b, buf_a, buf_b, buf_a, buf_b)
            sems = (sem_a, sem_b, sem_a, sem_b, sem_a, sem_b)
            cps = [None] * 6
            cps[0] = pltpu.async_copy(
                tabs[0].at[grp(irow[0])], bufs[0], sems[0])
            cps[1] = pltpu.async_copy(
                tabs[1].at[grp(irow[1])], bufs[1], sems[1])

            @pl.when(c >= 2)
            def _():
                pltpu.make_async_copy(
                    acc, out_hbm.at[pl.ds(base, ch)], sem_o).wait()
            cp_w = pltpu.async_copy(word_hbm.at[grp(0)], acc, sem_w)

            cp_w.wait()
            for j in range(6):
                cps[j].wait()
                buf = bufs[j]

                @pl.loop(0, ch)
                def _row(r):
                    for i in range(0, hdim, _LANES):
                        s = pl.ds(i, _LANES)
                        plsc.addupdate(acc.at[r, s], buf[r, s])

                if j + 2 < 6:
                    cps[j + 2] = pltpu.async_copy(
                        tabs[j + 2].at[grp(irow[j + 2])],
                        bufs[j + 2], sems[j + 2])

            pltpu.async_copy(acc, out_hbm.at[pl.ds(base, ch)], sem_o)

        @pl.loop(0, n_chunks, step=2)
        def _chunks(c):
            chunk_body(c, idx0, idx1, acc0, sem_o0, False)
            chunk_body(c + 1, idx1, idx0, acc1, sem_o1, True)

        for acc, sem_o in ((acc0, sem_o0), (acc1, sem_o1)):
            pltpu.make_async_copy(
                acc, out_hbm.at[pl.ds(w0, ch)], sem_o).wait()

    return k(word_emb, x_emb, y_emb, h_emb, w_emb, idx5c)


def _tc_finish(gsum, pos_emb, tids2, tt_pad, gamma2, beta2):
    tok, hdim = gsum.shape
    blk = 256
    n = tok // blk
    s_len = pos_emb.shape[0]
    pos_blocks = s_len // blk

    def body(g_ref, pos_ref, tid_ref, ttab_ref, gam_ref, bet_ref, o_ref):
        x = g_ref[...] + pos_ref[...]
        tid = tid_ref[...]
        x = x + jnp.where(tid < 1, ttab_ref[0:1, :], ttab_ref[1:2, :])
        mean = jnp.mean(x, axis=-1, keepdims=True)
        xc = x - mean
        var = jnp.mean(xc * xc, axis=-1, keepdims=True)
        o_ref[...] = xc * lax.rsqrt(var + _EPS) * gam_ref[...] + bet_ref[...]

    return pl.pallas_call(
        body,
        grid=(n,),
        in_specs=[
            pl.BlockSpec((blk, hdim), lambda i: (i, 0)),
            pl.BlockSpec((blk, hdim), lambda i: (i % pos_blocks, 0)),
            pl.BlockSpec((blk, 1), lambda i: (i, 0)),
            pl.BlockSpec((8, hdim), lambda i: (0, 0)),
            pl.BlockSpec((1, hdim), lambda i: (0, 0)),
            pl.BlockSpec((1, hdim), lambda i: (0, 0)),
        ],
        out_specs=pl.BlockSpec((blk, hdim), lambda i: (i, 0)),
        out_shape=jax.ShapeDtypeStruct((tok, hdim), jnp.float32),
    )(gsum, pos_emb, tids2, tt_pad, gamma2, beta2)


def kernel(input_ids, bbox, token_type_ids, word_emb, pos_emb,
           x_emb, y_emb, h_emb, w_emb, tt_emb, ln_gamma, ln_beta):
    b, s = input_ids.shape
    hdim = word_emb.shape[1]
    tok = b * s

    ids_flat = input_ids.reshape(tok)
    bbox_t = bbox.reshape(tok, 4).T
    nck = tok // _CH
    idx5c = (jnp.concatenate([ids_flat[None, :], bbox_t], axis=0)
             .reshape(5, nck, _CH).transpose(1, 0, 2))
    idx5c = jnp.concatenate(
        [idx5c, jnp.zeros((nck, 3, _CH), jnp.int32)], axis=1)
    idx5c = idx5c.reshape(nck, 8 * _CH)

    gsum = _sc_gather_sum(word_emb, x_emb, y_emb, h_emb, w_emb, idx5c, tok)

    tids2 = token_type_ids.reshape(tok, 1)
    tt_pad = jnp.zeros((8, hdim), tt_emb.dtype).at[:2, :].set(tt_emb)
    out = _tc_finish(gsum, pos_emb, tids2, tt_pad,
                     ln_gamma.reshape(1, hdim), ln_beta.reshape(1, hdim))
    return out.reshape(b, s, hdim)


# speedup vs baseline: 1.0025x; 1.0025x over previous
import functools

import jax
import jax.numpy as jnp
from jax import lax
from jax.experimental import pallas as pl
from jax.experimental.pallas import tpu as pltpu
from jax.experimental.pallas import tpu_sc as plsc

_EPS = 1e-12
_NC, _NS = 2, 16
_NW = _NC * _NS
_LANES = 16


_CH = 32


def _sc_gather_sum(word_emb, x_emb, y_emb, h_emb, w_emb, idx5c, tok):
    hdim = word_emb.shape[1]
    b_per_w = tok // _NW
    ch = _CH
    n_chunks = b_per_w // ch
    assert tok % _NW == 0 and b_per_w % (2 * ch) == 0 and hdim % _LANES == 0

    mesh = plsc.VectorSubcoreMesh(
        core_axis_name="c", subcore_axis_name="s",
        num_cores=_NC, num_subcores=_NS)

    @functools.partial(
        pl.kernel,
        out_type=jax.ShapeDtypeStruct((tok, hdim), jnp.float32),
        mesh=mesh,
        scratch_types=[
            pltpu.VMEM((8 * ch,), jnp.int32),
            pltpu.VMEM((8 * ch,), jnp.int32),
            pltpu.VMEM((ch, hdim), jnp.float32),
            pltpu.VMEM((ch, hdim), jnp.float32),
            pltpu.VMEM((ch, hdim), jnp.float32),
            pltpu.VMEM((ch, hdim), jnp.float32),
            pltpu.VMEM((ch, hdim), jnp.float32),
            pltpu.SemaphoreType.DMA,
            pltpu.SemaphoreType.DMA,
            pltpu.SemaphoreType.DMA,
            pltpu.SemaphoreType.DMA,
            pltpu.SemaphoreType.DMA,
            pltpu.SemaphoreType.DMA,
            pltpu.SemaphoreType.DMA,
        ],
    )
    def k(word_hbm, x_hbm, y_hbm, h_hbm, w_hbm, idx5_hbm, out_hbm,
          idx0, idx1, acc0, acc1, buf_a, buf_b, buf_c,
          sem_i, sem_w, sem_a, sem_b, sem_c, sem_o0, sem_o1):
        wid = lax.axis_index("s") * _NC + lax.axis_index("c")
        w0 = wid * b_per_w
        t0 = wid * n_chunks

        def fetch_idx(c, idxb):
            return pltpu.async_copy(idx5_hbm.at[t0 + c], idxb, sem_i)

        fetch_idx(0, idx0)

        def chunk_body(c, idxb, other_idxb, acc, sem_o, last_parity):
            base = w0 + c * ch

            def grp(j):
                return idxb.at[pl.ds(j * ch, ch)]

            pltpu.make_async_copy(idx5_hbm.at[t0 + c], idxb, sem_i).wait()
            if not last_parity:
                fetch_idx(c + 1, other_idxb)
            else:
                @pl.when(c + 1 < n_chunks)
                def _():
                    fetch_idx(c + 1, other_idxb)
            for i in range(0, ch, _LANES):
                idxb[pl.ds(5 * ch + i, _LANES)] = (
                    idxb[pl.ds(4 * ch + i, _LANES)]
                    - idxb[pl.ds(2 * ch + i, _LANES)])
                idxb[pl.ds(6 * ch + i, _LANES)] = (
                    idxb[pl.ds(3 * ch + i, _LANES)]
                    - idxb[pl.ds(1 * ch + i, _LANES)])

            tabs = (x_hbm, y_hbm, x_hbm, h_hbm, w_hbm, y_hbm)
            irow = (1, 2, 3, 5, 6, 4)
            bufs = (buf_a, buf_b, buf_c, buf_a, buf_b, buf_c)
            sems = (sem_a, sem_b, sem_c, sem_a, sem_b, sem_c)
            cps = [None] * 6
            for j0 in range(3):
                cps[j0] = pltpu.async_copy(
                    tabs[j0].at[grp(irow[j0])], bufs[j0], sems[j0])

            @pl.when(c >= 2)
            def _():
                pltpu.make_async_copy(
                    acc, out_hbm.at[pl.ds(base, ch)], sem_o).wait()
            cp_w = pltpu.async_copy(word_hbm.at[grp(0)], acc, sem_w)

            cp_w.wait()
            for j in range(6):
                cps[j].wait()
                buf = bufs[j]

                @pl.loop(0, ch)
                def _row(r):
                    for i in range(0, hdim, _LANES):
                        s = pl.ds(i, _LANES)
                        plsc.addupdate(acc.at[r, s], buf[r, s])

                if j + 3 < 6:
                    cps[j + 3] = pltpu.async_copy(
                        tabs[j + 3].at[grp(irow[j + 3])],
                        bufs[j + 3], sems[j + 3])

            pltpu.async_copy(acc, out_hbm.at[pl.ds(base, ch)], sem_o)

        @pl.loop(0, n_chunks, step=2)
        def _chunks(c):
            chunk_body(c, idx0, idx1, acc0, sem_o0, False)
            chunk_body(c + 1, idx1, idx0, acc1, sem_o1, True)

        for acc, sem_o in ((acc0, sem_o0), (acc1, sem_o1)):
            pltpu.make_async_copy(
                acc, out_hbm.at[pl.ds(w0, ch)], sem_o).wait()

    return k(word_emb, x_emb, y_emb, h_emb, w_emb, idx5c)


def _tc_finish(gsum, pos_emb, tids2, tt_pad, gamma2, beta2):
    tok, hdim = gsum.shape
    blk = 512
    n = tok // blk
    s_len = pos_emb.shape[0]
    pos_blocks = s_len // blk

    def body(g_ref, pos_ref, tid_ref, ttab_ref, gam_ref, bet_ref, o_ref):
        x = g_ref[...] + pos_ref[...]
        tid = tid_ref[...]
        x = x + jnp.where(tid < 1, ttab_ref[0:1, :], ttab_ref[1:2, :])
        mean = jnp.mean(x, axis=-1, keepdims=True)
        xc = x - mean
        var = jnp.mean(xc * xc, axis=-1, keepdims=True)
        o_ref[...] = xc * lax.rsqrt(var + _EPS) * gam_ref[...] + bet_ref[...]

    return pl.pallas_call(
        body,
        grid=(n,),
        in_specs=[
            pl.BlockSpec((blk, hdim), lambda i: (i, 0)),
            pl.BlockSpec((blk, hdim), lambda i: (i % pos_blocks, 0)),
            pl.BlockSpec((blk, 1), lambda i: (i, 0)),
            pl.BlockSpec((8, hdim), lambda i: (0, 0)),
            pl.BlockSpec((1, hdim), lambda i: (0, 0)),
            pl.BlockSpec((1, hdim), lambda i: (0, 0)),
        ],
        out_specs=pl.BlockSpec((blk, hdim), lambda i: (i, 0)),
        out_shape=jax.ShapeDtypeStruct((tok, hdim), jnp.float32),
    )(gsum, pos_emb, tids2, tt_pad, gamma2, beta2)


def kernel(input_ids, bbox, token_type_ids, word_emb, pos_emb,
           x_emb, y_emb, h_emb, w_emb, tt_emb, ln_gamma, ln_beta):
    b, s = input_ids.shape
    hdim = word_emb.shape[1]
    tok = b * s

    ids_flat = input_ids.reshape(tok)
    bbox_t = bbox.reshape(tok, 4).T
    nck = tok // _CH
    idx5c = (jnp.concatenate([ids_flat[None, :], bbox_t], axis=0)
             .reshape(5, nck, _CH).transpose(1, 0, 2))
    idx5c = jnp.concatenate(
        [idx5c, jnp.zeros((nck, 3, _CH), jnp.int32)], axis=1)
    idx5c = idx5c.reshape(nck, 8 * _CH)

    gsum = _sc_gather_sum(word_emb, x_emb, y_emb, h_emb, w_emb, idx5c, tok)

    tids2 = token_type_ids.reshape(tok, 1)
    tt_pad = jnp.zeros((8, hdim), tt_emb.dtype).at[:2, :].set(tt_emb)
    out = _tc_finish(gsum, pos_emb, tids2, tt_pad,
                     ln_gamma.reshape(1, hdim), ln_beta.reshape(1, hdim))
    return out.reshape(b, s, hdim)
